# baseline (speedup 1.0000x reference)
import functools

import jax
import jax.numpy as jnp
from jax.experimental import pallas as pl
from jax.experimental.pallas import tpu as pltpu

B, N, E, D, H, NEF = 1, 10000, 320000, 128, 128, 140

EDGE_BLK = 2560


def _edge_mlp_body(ea_ref, w1_ref, b1_ref, w2_ref, b2_ref, out_ref):
    x = ea_ref[...]
    h = jnp.dot(x, w1_ref[...], preferred_element_type=jnp.float32) + b1_ref[...]
    h = h * jax.nn.sigmoid(h)
    out_ref[...] = (
        jnp.dot(h, w2_ref[...], preferred_element_type=jnp.float32) + b2_ref[...]
    )


def _edge_mlp(edge_attr2d, W1, b1, W2, b2):
    grid = (E // EDGE_BLK,)
    return pl.pallas_call(
        _edge_mlp_body,
        grid=grid,
        in_specs=[
            pl.BlockSpec((EDGE_BLK, NEF), lambda i: (i, 0)),
            pl.BlockSpec((NEF, H), lambda i: (0, 0)),
            pl.BlockSpec((1, H), lambda i: (0, 0)),
            pl.BlockSpec((H, H), lambda i: (0, 0)),
            pl.BlockSpec((1, H), lambda i: (0, 0)),
        ],
        out_specs=pl.BlockSpec((EDGE_BLK, H), lambda i: (i, 0)),
        out_shape=jax.ShapeDtypeStruct((E, H), jnp.float32),
    )(edge_attr2d, W1, b1.reshape(1, H), W2, b2.reshape(1, H))


def _final_body(s_ref, c_ref, x_ref, wres_ref, bres_ref, wo1_ref, bo1_ref,
                wo2_ref, bo2_ref, wo3_ref, bo3_ref, out_ref):
    s = s_ref[...]
    c = c_ref[...]
    agg = s / jnp.maximum(c, 1.0)
    aug = agg + (
        jnp.dot(x_ref[...], wres_ref[...], preferred_element_type=jnp.float32)
        + bres_ref[...]
    )
    h = jnp.maximum(
        jnp.dot(aug, wo1_ref[...], preferred_element_type=jnp.float32)
        + bo1_ref[...], 0.0)
    h = jnp.maximum(
        jnp.dot(h, wo2_ref[...], preferred_element_type=jnp.float32)
        + bo2_ref[...], 0.0)
    out_ref[...] = (
        jnp.dot(h, wo3_ref[...], preferred_element_type=jnp.float32)
        + bo3_ref[...]
    )


def _final_mlp(sums, counts2d, x2d, Wres, bres, Wo1, bo1, Wo2, bo2, Wo3, bo3):
    NBLK = 2000
    grid = (N // NBLK,)
    wspec = pl.BlockSpec((H, H), lambda i: (0, 0))
    bspec = pl.BlockSpec((1, H), lambda i: (0, 0))
    return pl.pallas_call(
        _final_body,
        grid=grid,
        in_specs=[
            pl.BlockSpec((NBLK, H), lambda i: (i, 0)),
            pl.BlockSpec((NBLK, 1), lambda i: (i, 0)),
            pl.BlockSpec((NBLK, D), lambda i: (i, 0)),
            pl.BlockSpec((D, H), lambda i: (0, 0)), bspec,
            wspec, bspec, wspec, bspec,
            pl.BlockSpec((H, D), lambda i: (0, 0)),
            pl.BlockSpec((1, D), lambda i: (0, 0)),
        ],
        out_specs=pl.BlockSpec((NBLK, D), lambda i: (i, 0)),
        out_shape=jax.ShapeDtypeStruct((N, D), jnp.float32),
    )(sums, counts2d, x2d, Wres, bres.reshape(1, H), Wo1, bo1.reshape(1, H),
      Wo2, bo2.reshape(1, H), Wo3, bo3.reshape(1, D))


def kernel(inputs, edge_attr, edges, W1, b1, W2, b2, Wres, bres,
           Wo1, bo1, Wo2, bo2, Wo3, bo3):
    edge_attr2d = edge_attr.reshape(E, NEF)
    msgs = _edge_mlp(edge_attr2d, W1, b1, W2, b2)

    dst = edges[1]
    sums = jax.ops.segment_sum(msgs, dst, num_segments=N)
    counts = jax.ops.segment_sum(jnp.ones((E,), jnp.float32), dst, num_segments=N)

    pred = _final_mlp(sums, counts.reshape(N, 1), inputs.reshape(N, D),
                      Wres, bres, Wo1, bo1, Wo2, bo2, Wo3, bo3)
    return pred.reshape(B, N, D)


# speedup vs baseline: 2.4283x; 2.4283x over previous
import functools

import jax
import jax.numpy as jnp
from jax import lax
from jax.experimental import pallas as pl
from jax.experimental.pallas import tpu as pltpu
from jax.experimental.pallas import tpu_sc as plsc

B, N, E, D, H, NEF = 1, 10000, 320000, 128, 128, 140

EDGE_BLK = 2560

NC, NS = 2, 16
NT = NC * NS
EPT = E // NT
BATCH = 80
NB = EPT // BATCH
SUBB = 5
NG = NB // SUBB
NP = 10240
ZR = NP // NS
ZSTEPS = ZR // BATCH
CW = 128


def _edge_mlp_body(ea_ref, w1_ref, b1_ref, w2_ref, b2_ref, out_ref):
    x = ea_ref[...]
    h = jnp.dot(x, w1_ref[...], preferred_element_type=jnp.float32) + b1_ref[...]
    h = h * jax.nn.sigmoid(h)
    out_ref[...] = (
        jnp.dot(h, w2_ref[...], preferred_element_type=jnp.float32) + b2_ref[...]
    )


def _edge_mlp(edge_attr2d, W1, b1, W2, b2):
    grid = (E // EDGE_BLK,)
    return pl.pallas_call(
        _edge_mlp_body,
        grid=grid,
        in_specs=[
            pl.BlockSpec((EDGE_BLK, NEF), lambda i: (i, 0)),
            pl.BlockSpec((NEF, H), lambda i: (0, 0)),
            pl.BlockSpec((1, H), lambda i: (0, 0)),
            pl.BlockSpec((H, H), lambda i: (0, 0)),
            pl.BlockSpec((1, H), lambda i: (0, 0)),
        ],
        out_specs=pl.BlockSpec((EDGE_BLK, H), lambda i: (i, 0)),
        out_shape=jax.ShapeDtypeStruct((E, H), jnp.float32),
    )(edge_attr2d, W1, b1.reshape(1, H), W2, b2.reshape(1, H))


def _sc_sums(msgs, dst2d, zeros_h):
    mesh = plsc.VectorSubcoreMesh(core_axis_name="c", subcore_axis_name="s")

    @functools.partial(
        pl.kernel,
        mesh=mesh,
        out_type=jax.ShapeDtypeStruct((NC, NP, H), jnp.float32),
        scratch_types=[
            pltpu.VMEM((SUBB, BATCH), jnp.int32),
            pltpu.VMEM((BATCH, H), jnp.float32),
            pltpu.VMEM_SHARED((NP, H), jnp.float32),
        ],
    )
    def k(msgs_hbm, dst_hbm, zh_hbm, sums_out, idx_v, rows_v, sums_sh):
        c = lax.axis_index("c")
        s = lax.axis_index("s")
        wid = s * NC + c
        ebase = wid * EPT
        zr = s * ZR

        pltpu.sync_copy(zh_hbm, rows_v)
        for i in range(ZSTEPS):
            pltpu.sync_copy(rows_v, sums_sh.at[pl.ds(zr + i * BATCH, BATCH)])
        plsc.subcore_barrier()

        def group_body(g, carry):
            pltpu.sync_copy(dst_hbm.at[wid, g], idx_v)
            for j in range(SUBB):
                kk = g * SUBB + j
                pltpu.sync_copy(msgs_hbm.at[pl.ds(ebase + kk * BATCH, BATCH)],
                                rows_v)
                pltpu.sync_copy(rows_v, sums_sh.at[idx_v.at[j]], add=True)
            return carry

        lax.fori_loop(0, NG, group_body, 0)
        plsc.subcore_barrier()

        for i in range(ZSTEPS):
            pltpu.sync_copy(sums_sh.at[pl.ds(zr + i * BATCH, BATCH)], rows_v)
            pltpu.sync_copy(rows_v, sums_out.at[c, pl.ds(zr + i * BATCH, BATCH)])

    return k(msgs, dst2d, zeros_h)


def _sc_counts(dst2d, zeros_c, ones_c):
    mesh = plsc.VectorSubcoreMesh(core_axis_name="c", subcore_axis_name="s")

    @functools.partial(
        pl.kernel,
        mesh=mesh,
        out_type=jax.ShapeDtypeStruct((NC, NP, CW), jnp.float32),
        scratch_types=[
            pltpu.VMEM((SUBB, BATCH), jnp.int32),
            pltpu.VMEM((BATCH, CW), jnp.float32),
            pltpu.VMEM((BATCH, CW), jnp.float32),
            pltpu.VMEM_SHARED((NP, CW), jnp.float32),
        ],
    )
    def k(dst_hbm, zc_hbm, ones_hbm, cnts_out, idx_v, ones_v, cst_v, cnts_sh):
        c = lax.axis_index("c")
        s = lax.axis_index("s")
        wid = s * NC + c
        zr = s * ZR

        pltpu.sync_copy(zc_hbm, cst_v)
        pltpu.sync_copy(ones_hbm, ones_v)
        for i in range(ZSTEPS):
            pltpu.sync_copy(cst_v, cnts_sh.at[pl.ds(zr + i * BATCH, BATCH)])
        plsc.subcore_barrier()

        def group_body(g, carry):
            pltpu.sync_copy(dst_hbm.at[wid, g], idx_v)
            for j in range(SUBB):
                pltpu.sync_copy(ones_v, cnts_sh.at[idx_v.at[j]], add=True)
            return carry

        lax.fori_loop(0, NG, group_body, 0)
        plsc.subcore_barrier()

        for i in range(ZSTEPS):
            pltpu.sync_copy(cnts_sh.at[pl.ds(zr + i * BATCH, BATCH)], cst_v)
            pltpu.sync_copy(cst_v, cnts_out.at[c, pl.ds(zr + i * BATCH, BATCH)])

    return k(dst2d, zeros_c, ones_c)


def _final_body(s0_ref, s1_ref, c0_ref, c1_ref, x_ref, wres_ref, bres_ref,
                wo1_ref, bo1_ref, wo2_ref, bo2_ref, wo3_ref, bo3_ref, out_ref):
    sums = s0_ref[...] + s1_ref[...]
    cnt = c0_ref[...] + c1_ref[...]
    agg = sums / jnp.maximum(cnt, 1.0)
    aug = agg + (
        jnp.dot(x_ref[...], wres_ref[...], preferred_element_type=jnp.float32)
        + bres_ref[...]
    )
    h = jnp.maximum(
        jnp.dot(aug, wo1_ref[...], preferred_element_type=jnp.float32)
        + bo1_ref[...], 0.0)
    h = jnp.maximum(
        jnp.dot(h, wo2_ref[...], preferred_element_type=jnp.float32)
        + bo2_ref[...], 0.0)
    out_ref[...] = (
        jnp.dot(h, wo3_ref[...], preferred_element_type=jnp.float32)
        + bo3_ref[...]
    )


def _final_mlp(s0, s1, c0, c1, x2d, Wres, bres, Wo1, bo1, Wo2, bo2, Wo3, bo3):
    NBLK = 2000
    grid = (N // NBLK,)
    wspec = pl.BlockSpec((H, H), lambda i: (0, 0))
    bspec = pl.BlockSpec((1, H), lambda i: (0, 0))
    return pl.pallas_call(
        _final_body,
        grid=grid,
        in_specs=[
            pl.BlockSpec((NBLK, H), lambda i: (i, 0)),
            pl.BlockSpec((NBLK, H), lambda i: (i, 0)),
            pl.BlockSpec((NBLK, 1), lambda i: (i, 0)),
            pl.BlockSpec((NBLK, 1), lambda i: (i, 0)),
            pl.BlockSpec((NBLK, D), lambda i: (i, 0)),
            pl.BlockSpec((D, H), lambda i: (0, 0)), bspec,
            wspec, bspec, wspec, bspec,
            pl.BlockSpec((H, D), lambda i: (0, 0)),
            pl.BlockSpec((1, D), lambda i: (0, 0)),
        ],
        out_specs=pl.BlockSpec((NBLK, D), lambda i: (i, 0)),
        out_shape=jax.ShapeDtypeStruct((N, D), jnp.float32),
    )(s0, s1, c0, c1, x2d, Wres, bres.reshape(1, H), Wo1, bo1.reshape(1, H),
      Wo2, bo2.reshape(1, H), Wo3, bo3.reshape(1, D))


def kernel(inputs, edge_attr, edges, W1, b1, W2, b2, Wres, bres,
           Wo1, bo1, Wo2, bo2, Wo3, bo3):
    edge_attr2d = edge_attr.reshape(E, NEF)
    msgs = _edge_mlp(edge_attr2d, W1, b1, W2, b2)

    dst2d = edges[1].reshape(NT, NG, SUBB, BATCH)
    zeros_h = jnp.zeros((BATCH, H), jnp.float32)
    zeros_c = jnp.zeros((BATCH, CW), jnp.float32)
    ones_c = jnp.ones((BATCH, CW), jnp.float32)
    cnts = _sc_counts(dst2d, zeros_c, ones_c)
    sums = _sc_sums(msgs, dst2d, zeros_h)

    pred = _final_mlp(sums[0, :N], sums[1, :N], cnts[0, :N, 0:1],
                      cnts[1, :N, 0:1], inputs.reshape(N, D),
                      Wres, bres, Wo1, bo1, Wo2, bo2, Wo3, bo3)
    return pred.reshape(B, N, D)
